# baseline (speedup 1.0000x reference)
import jax
import jax.numpy as jnp
from jax.experimental import pallas as pl
from jax.experimental.pallas import tpu as pltpu

N = 4096
R = 512
GRID = N // R
F32 = jnp.float32
BF16 = jnp.bfloat16
TDIMS = (((0,), (0,)), ((), ()))


def _p1(adj_ref, x_ref, w0_ref, adj8_ref, d1_ref, cs_ref, v1_ref):
    blk = adj_ref[...]
    adj8_ref[...] = blk.astype(jnp.int8)
    rs = jnp.sum(blk, axis=1, keepdims=True)
    d1 = (rs + 1.0) ** -0.5
    d1_ref[...] = d1

    @pl.when(pl.program_id(0) == 0)
    def _():
        cs_ref[...] = jnp.zeros_like(cs_ref)

    cs_ref[...] += jnp.sum(blk, axis=0, keepdims=True)
    xw = jnp.dot(x_ref[...], w0_ref[...], preferred_element_type=F32)
    v1_ref[...] = (d1 * xw).astype(BF16)


def _p2(adj_ref, v1f_ref, v1b_ref, d1_ref, b0_ref, w1_ref, v2_ref):
    acc = jnp.dot(adj_ref[...].astype(BF16), v1f_ref[...],
                  preferred_element_type=F32)
    pre = acc + v1b_ref[...].astype(F32)
    h1 = jax.nn.relu(d1_ref[...] * pre + b0_ref[...])
    v2_ref[...] = d1_ref[...] * jnp.dot(h1, w1_ref[...],
                                        preferred_element_type=F32)


def _p3(adj_ref, v2f_ref, v2b_ref, d1_ref, d2_ref, b1_ref, wg1_ref,
        logits_ref, v3_ref, acc4_ref):
    blk = adj_ref[...].astype(F32)
    acc = jnp.dot(blk, v2f_ref[...], preferred_element_type=F32)
    logits = d1_ref[...] * (acc + v2b_ref[...]) + b1_ref[...]
    logits_ref[...] = logits
    v3 = d2_ref[...] * jnp.dot(logits, wg1_ref[...],
                               preferred_element_type=F32)
    v3_ref[...] = v3

    @pl.when(pl.program_id(0) == 0)
    def _():
        acc4_ref[...] = jnp.zeros_like(acc4_ref)

    acc4_ref[...] += jax.lax.dot_general(blk, v3, TDIMS,
                                         preferred_element_type=F32)


def _p4(adj_ref, acc4b_ref, v3b_ref, d2_ref, bg1_ref, wg2_ref,
        v4_ref, acc5_ref):
    t = jax.nn.relu(d2_ref[...] * (acc4b_ref[...] + v3b_ref[...])
                    + bg1_ref[...])
    v4 = d2_ref[...] * jnp.dot(t, wg2_ref[...], preferred_element_type=F32)
    v4_ref[...] = v4

    @pl.when(pl.program_id(0) == 0)
    def _():
        acc5_ref[...] = jnp.zeros_like(acc5_ref)

    acc5_ref[...] += jax.lax.dot_general(adj_ref[...].astype(F32), v4, TDIMS,
                                         preferred_element_type=F32)


def _p5(acc5_ref, v4f_ref, d2f_ref, bg2_ref, logitsf_ref, out_ref):
    t2 = d2f_ref[...] * (acc5_ref[...] + v4f_ref[...]) + bg2_ref[...]
    t3 = jnp.log(jnp.exp(t2) + 1.1)
    o = logitsf_ref[...] * t3
    m = jnp.max(o, axis=1, keepdims=True)
    lse = m + jnp.log(jnp.sum(jnp.exp(o - m), axis=1, keepdims=True))
    out_ref[...] = o - lse


def _row_blk(f):
    return pl.BlockSpec((R, f), lambda i: (i, 0))


def _full(n, f):
    return pl.BlockSpec((n, f), lambda i: (0, 0))


@jax.jit
def kernel(x, adj, W0, b0, W1, b1, Wg1, bg1, Wg2, bg2):
    D = x.shape[1]
    H = W0.shape[1]
    C = W1.shape[1]
    b0r, b1r = b0[None, :], b1[None, :]
    bg1r, bg2r = bg1[None, :], bg2[None, :]

    adj8, d1, cs, v1 = pl.pallas_call(
        _p1,
        grid=(GRID,),
        in_specs=[_row_blk(N), _row_blk(D), _full(D, H)],
        out_specs=[_row_blk(N), _row_blk(1), _full(1, N), _row_blk(H)],
        out_shape=[jax.ShapeDtypeStruct((N, N), jnp.int8),
                   jax.ShapeDtypeStruct((N, 1), F32),
                   jax.ShapeDtypeStruct((1, N), F32),
                   jax.ShapeDtypeStruct((N, H), BF16)],
    )(adj, x, W0)

    d2 = (cs.reshape(N, 1) + 1.0) ** -0.5

    v2 = pl.pallas_call(
        _p2,
        grid=(GRID,),
        in_specs=[_row_blk(N), _full(N, H), _row_blk(H), _row_blk(1),
                  _full(1, H), _full(H, C)],
        out_specs=_row_blk(C),
        out_shape=jax.ShapeDtypeStruct((N, C), F32),
    )(adj8, v1, v1, d1, b0r, W1)

    logits, v3, acc4 = pl.pallas_call(
        _p3,
        grid=(GRID,),
        in_specs=[_row_blk(N), _full(N, C), _row_blk(C), _row_blk(1),
                  _row_blk(1), _full(1, C), _full(C, C)],
        out_specs=[_row_blk(C), _row_blk(C), _full(N, C)],
        out_shape=[jax.ShapeDtypeStruct((N, C), F32),
                   jax.ShapeDtypeStruct((N, C), F32),
                   jax.ShapeDtypeStruct((N, C), F32)],
    )(adj8, v2, v2, d1, d2, b1r, Wg1)

    v4, acc5 = pl.pallas_call(
        _p4,
        grid=(GRID,),
        in_specs=[_row_blk(N), _row_blk(C), _row_blk(C), _row_blk(1),
                  _full(1, C), _full(C, C)],
        out_specs=[_row_blk(C), _full(N, C)],
        out_shape=[jax.ShapeDtypeStruct((N, C), F32),
                   jax.ShapeDtypeStruct((N, C), F32)],
    )(adj8, acc4, v3, d2, bg1r, Wg2)

    out = pl.pallas_call(
        _p5,
        grid=(1,),
        in_specs=[_full(N, C), _full(N, C), _full(N, 1), _full(1, C),
                  _full(N, C)],
        out_specs=_full(N, C),
        out_shape=jax.ShapeDtypeStruct((N, C), F32),
    )(acc5, v4, d2, bg2r, logits)

    return out


# speedup vs baseline: 1.0510x; 1.0510x over previous
import jax
import jax.numpy as jnp
from jax.experimental import pallas as pl
from jax.experimental.pallas import tpu as pltpu

N = 4096
R = 512
GRID = N // R
F32 = jnp.float32
BF16 = jnp.bfloat16
TDIMS = (((0,), (0,)), ((), ()))


def _p1(adj_ref, x_ref, w0_ref, adj8_ref, d1_ref, cs_ref, v1_ref):
    blk = adj_ref[...]
    adj8_ref[...] = blk.astype(jnp.int8)
    rs = jnp.sum(blk, axis=1, keepdims=True)
    d1 = (rs + 1.0) ** -0.5
    d1_ref[...] = d1

    @pl.when(pl.program_id(0) == 0)
    def _():
        cs_ref[...] = jnp.zeros_like(cs_ref)

    cs_ref[...] += jnp.sum(blk, axis=0, keepdims=True)
    xw = jnp.dot(x_ref[...], w0_ref[...], preferred_element_type=F32)
    v1_ref[...] = (d1 * xw).astype(BF16)


def _p2(adj_ref, v1f_ref, v1b_ref, d1_ref, b0_ref, w1_ref, v2_ref):
    acc = jnp.dot(adj_ref[...].astype(BF16), v1f_ref[...],
                  preferred_element_type=F32)
    pre = acc + v1b_ref[...].astype(F32)
    h1 = jax.nn.relu(d1_ref[...] * pre + b0_ref[...])
    v2_ref[...] = d1_ref[...] * jnp.dot(h1, w1_ref[...],
                                        preferred_element_type=F32)


def _p3(adj_ref, v2f_ref, v2b_ref, d1_ref, d2_ref, b1_ref, wg1_ref,
        logits_ref, v3_ref, acc4_ref):
    blk = adj_ref[...].astype(BF16)
    acc = jnp.dot(blk, v2f_ref[...].astype(BF16),
                  preferred_element_type=F32)
    logits = d1_ref[...] * (acc + v2b_ref[...]) + b1_ref[...]
    logits_ref[...] = logits
    v3 = d2_ref[...] * jnp.dot(logits, wg1_ref[...],
                               preferred_element_type=F32)
    v3_ref[...] = v3

    @pl.when(pl.program_id(0) == 0)
    def _():
        acc4_ref[...] = jnp.zeros_like(acc4_ref)

    acc4_ref[...] += jax.lax.dot_general(blk, v3.astype(BF16), TDIMS,
                                         preferred_element_type=F32)


def _p4(adj_ref, acc4b_ref, v3b_ref, d2_ref, d2f_ref, bg1_ref, wg2_ref,
        bg2_ref, logitsf_ref, out_ref, v4s_ref, acc5_ref):
    k = pl.program_id(0)
    t = jax.nn.relu(d2_ref[...] * (acc4b_ref[...] + v3b_ref[...])
                    + bg1_ref[...])
    v4 = d2_ref[...] * jnp.dot(t, wg2_ref[...], preferred_element_type=F32)
    v4s_ref[pl.ds(k * R, R), :] = v4

    @pl.when(k == 0)
    def _():
        acc5_ref[...] = jnp.zeros_like(acc5_ref)

    acc5_ref[...] += jax.lax.dot_general(adj_ref[...].astype(BF16),
                                         v4.astype(BF16), TDIMS,
                                         preferred_element_type=F32)

    @pl.when(k == GRID - 1)
    def _():
        t2 = d2f_ref[...] * (acc5_ref[...] + v4s_ref[...]) + bg2_ref[...]
        t3 = jnp.log(jnp.exp(t2) + 1.1)
        o = logitsf_ref[...] * t3
        m = jnp.max(o, axis=1, keepdims=True)
        lse = m + jnp.log(jnp.sum(jnp.exp(o - m), axis=1, keepdims=True))
        out_ref[...] = o - lse


def _row_blk(f):
    return pl.BlockSpec((R, f), lambda i: (i, 0))


def _full(n, f):
    return pl.BlockSpec((n, f), lambda i: (0, 0))


@jax.jit
def kernel(x, adj, W0, b0, W1, b1, Wg1, bg1, Wg2, bg2):
    D = x.shape[1]
    H = W0.shape[1]
    C = W1.shape[1]
    b0r, b1r = b0[None, :], b1[None, :]
    bg1r, bg2r = bg1[None, :], bg2[None, :]

    adj8, d1, cs, v1 = pl.pallas_call(
        _p1,
        grid=(GRID,),
        in_specs=[_row_blk(N), _row_blk(D), _full(D, H)],
        out_specs=[_row_blk(N), _row_blk(1), _full(1, N), _row_blk(H)],
        out_shape=[jax.ShapeDtypeStruct((N, N), jnp.int8),
                   jax.ShapeDtypeStruct((N, 1), F32),
                   jax.ShapeDtypeStruct((1, N), F32),
                   jax.ShapeDtypeStruct((N, H), BF16)],
    )(adj, x, W0)

    d2 = (cs.reshape(N, 1) + 1.0) ** -0.5

    v2 = pl.pallas_call(
        _p2,
        grid=(GRID,),
        in_specs=[_row_blk(N), _full(N, H), _row_blk(H), _row_blk(1),
                  _full(1, H), _full(H, C)],
        out_specs=_row_blk(C),
        out_shape=jax.ShapeDtypeStruct((N, C), F32),
    )(adj8, v1, v1, d1, b0r, W1)

    logits, v3, acc4 = pl.pallas_call(
        _p3,
        grid=(GRID,),
        in_specs=[_row_blk(N), _full(N, C), _row_blk(C), _row_blk(1),
                  _row_blk(1), _full(1, C), _full(C, C)],
        out_specs=[_row_blk(C), _row_blk(C), _full(N, C)],
        out_shape=[jax.ShapeDtypeStruct((N, C), F32),
                   jax.ShapeDtypeStruct((N, C), F32),
                   jax.ShapeDtypeStruct((N, C), F32)],
    )(adj8, v2, v2, d1, d2, b1r, Wg1)

    out = pl.pallas_call(
        _p4,
        grid=(GRID,),
        in_specs=[_row_blk(N), _row_blk(C), _row_blk(C), _row_blk(1),
                  _full(N, 1), _full(1, C), _full(C, C), _full(1, C),
                  _full(N, C)],
        out_specs=_full(N, C),
        out_shape=jax.ShapeDtypeStruct((N, C), F32),
        scratch_shapes=[pltpu.VMEM((N, C), F32), pltpu.VMEM((N, C), F32)],
    )(adj8, acc4, v3, d2, d2, bg1r, Wg2, bg2r, logits)

    return out


# speedup vs baseline: 1.5036x; 1.4306x over previous
import jax
import jax.numpy as jnp
from jax.experimental import pallas as pl
from jax.experimental.pallas import tpu as pltpu

N = 4096
R = 512
GRID = N // R
F32 = jnp.float32
BF16 = jnp.bfloat16
TDIMS = (((0,), (0,)), ((), ()))


def _mega(adj_ref, x_ref, w0_ref, b0_ref, w1_ref, b1_ref, wg1_ref, bg1_ref,
          wg2_ref, bg2_ref, out_ref,
          adj8v, d1s, css, d2s, v1s, v2s, logits_s, v3s, acc4t, v4s, acc5t):
    i = pl.program_id(0)
    stage = i // GRID
    k = i % GRID
    sl = pl.ds(k * R, R)

    @pl.when(stage == 0)
    def _():
        blk = adj_ref[...]
        adj8v[sl, :] = blk.astype(jnp.int8)
        rs = jnp.sum(blk, axis=1, keepdims=True)
        d1 = (rs + 1.0) ** -0.5
        d1s[sl, :] = d1

        @pl.when(k == 0)
        def _():
            css[...] = jnp.zeros_like(css)

        css[...] += jnp.sum(blk, axis=0, keepdims=True)
        xw = jnp.dot(x_ref[...], w0_ref[...], preferred_element_type=F32)
        v1s[sl, :] = (d1 * xw).astype(BF16)

    @pl.when(stage == 1)
    def _():
        @pl.when(k == 0)
        def _():
            d2s[...] = (css[...].T + 1.0) ** -0.5

        blk = adj8v[sl, :].astype(BF16)
        acc = jnp.dot(blk, v1s[...], preferred_element_type=F32)
        pre = acc + v1s[sl, :].astype(F32)
        d1 = d1s[sl, :]
        h1 = jax.nn.relu(d1 * pre + b0_ref[...])
        v2s[sl, :] = d1 * jnp.dot(h1, w1_ref[...], preferred_element_type=F32)

    @pl.when(stage == 2)
    def _():
        blk = adj8v[sl, :].astype(BF16)
        acc = jnp.dot(blk, v2s[...].astype(BF16), preferred_element_type=F32)
        logits = d1s[sl, :] * (acc + v2s[sl, :]) + b1_ref[...]
        logits_s[sl, :] = logits
        v3 = d2s[sl, :] * jnp.dot(logits, wg1_ref[...],
                                  preferred_element_type=F32)
        v3s[sl, :] = v3

        @pl.when(k == 0)
        def _():
            acc4t[...] = jnp.zeros_like(acc4t)

        acc4t[...] += jax.lax.dot_general(v3.astype(BF16), blk, TDIMS,
                                          preferred_element_type=F32)

    @pl.when(stage == 3)
    def _():
        blk = adj8v[sl, :].astype(BF16)
        acc4b = acc4t[:, sl].T
        t = jax.nn.relu(d2s[sl, :] * (acc4b + v3s[sl, :]) + bg1_ref[...])
        v4 = d2s[sl, :] * jnp.dot(t, wg2_ref[...], preferred_element_type=F32)
        v4s[sl, :] = v4

        @pl.when(k == 0)
        def _():
            acc5t[...] = jnp.zeros_like(acc5t)

        acc5t[...] += jax.lax.dot_general(v4.astype(BF16), blk, TDIMS,
                                          preferred_element_type=F32)

        @pl.when(k == GRID - 1)
        def _():
            t2 = d2s[...] * (acc5t[...].T + v4s[...]) + bg2_ref[...]
            t3 = jnp.log(jnp.exp(t2) + 1.1)
            o = logits_s[...] * t3
            m = jnp.max(o, axis=1, keepdims=True)
            lse = m + jnp.log(jnp.sum(jnp.exp(o - m), axis=1, keepdims=True))
            out_ref[...] = o - lse


@jax.jit
def kernel(x, adj, W0, b0, W1, b1, Wg1, bg1, Wg2, bg2):
    D = x.shape[1]
    H = W0.shape[1]
    C = W1.shape[1]

    def strip(f):
        return pl.BlockSpec((R, f),
                            lambda i: (jnp.minimum(i, GRID - 1), 0))

    def full(n, f):
        return pl.BlockSpec((n, f), lambda i: (0, 0))

    out = pl.pallas_call(
        _mega,
        grid=(4 * GRID,),
        in_specs=[strip(N), strip(D), full(D, H), full(1, H), full(H, C),
                  full(1, C), full(C, C), full(1, C), full(C, C),
                  full(1, C)],
        out_specs=full(N, C),
        out_shape=jax.ShapeDtypeStruct((N, C), F32),
        scratch_shapes=[
            pltpu.VMEM((N, N), jnp.int8),
            pltpu.VMEM((N, 1), F32),
            pltpu.VMEM((1, N), F32),
            pltpu.VMEM((N, 1), F32),
            pltpu.VMEM((N, H), BF16),
            pltpu.VMEM((N, C), F32),
            pltpu.VMEM((N, C), F32),
            pltpu.VMEM((N, C), F32),
            pltpu.VMEM((C, N), F32),
            pltpu.VMEM((N, C), F32),
            pltpu.VMEM((C, N), F32),
        ],
        compiler_params=pltpu.CompilerParams(
            vmem_limit_bytes=100 * 1024 * 1024),
    )(adj, x, W0, b0[None, :], W1, b1[None, :], Wg1, bg1[None, :], Wg2,
      bg2[None, :])

    return out


# speedup vs baseline: 1.5261x; 1.0149x over previous
import jax
import jax.numpy as jnp
from jax.experimental import pallas as pl
from jax.experimental.pallas import tpu as pltpu

N = 4096
R0 = 256
NIN = N // R0
RC = 1024
NC = N // RC
F32 = jnp.float32
BF16 = jnp.bfloat16
TDIMS = (((0,), (0,)), ((), ()))


def _mega(adj_ref, x_ref, w0_ref, b0_ref, w1_ref, b1_ref, wg1_ref, bg1_ref,
          wg2_ref, bg2_ref, out_ref,
          adj8v, d1s, css, d2s, v1s, v2s, logits_s, v3s, acc4t, v4s, acc5t):
    i = pl.program_id(0)

    @pl.when(i < NIN)
    def _():
        sl = pl.ds(i * R0, R0)
        blk = adj_ref[...]
        adj8v[sl, :] = blk.astype(jnp.int8)
        rs = jnp.sum(blk, axis=1, keepdims=True)
        d1 = (rs + 1.0) ** -0.5
        d1s[sl, :] = d1

        @pl.when(i == 0)
        def _():
            css[...] = jnp.zeros_like(css)

        css[...] += jnp.sum(blk, axis=0, keepdims=True)
        xw = jnp.dot(x_ref[...], w0_ref[...], preferred_element_type=F32)
        v1s[sl, :] = (d1 * xw).astype(BF16)

    @pl.when(jnp.logical_and(i >= NIN, i < NIN + NC))
    def _():
        k = i - NIN
        sl = pl.ds(k * RC, RC)

        @pl.when(k == 0)
        def _():
            d2s[...] = (css[...].T + 1.0) ** -0.5

        blk = adj8v[sl, :].astype(BF16)
        acc = jnp.dot(blk, v1s[...], preferred_element_type=F32)
        pre = acc + v1s[sl, :].astype(F32)
        d1 = d1s[sl, :]
        h1 = jax.nn.relu(d1 * pre + b0_ref[...])
        v2s[sl, :] = d1 * jnp.dot(h1, w1_ref[...], preferred_element_type=F32)

    @pl.when(jnp.logical_and(i >= NIN + NC, i < NIN + 2 * NC))
    def _():
        k = i - (NIN + NC)
        sl = pl.ds(k * RC, RC)
        blk = adj8v[sl, :].astype(BF16)
        acc = jnp.dot(blk, v2s[...].astype(BF16), preferred_element_type=F32)
        logits = d1s[sl, :] * (acc + v2s[sl, :]) + b1_ref[...]
        logits_s[sl, :] = logits
        v3 = d2s[sl, :] * jnp.dot(logits, wg1_ref[...],
                                  preferred_element_type=F32)
        v3s[sl, :] = v3

        @pl.when(k == 0)
        def _():
            acc4t[...] = jnp.zeros_like(acc4t)

        acc4t[...] += jax.lax.dot_general(v3.astype(BF16), blk, TDIMS,
                                          preferred_element_type=F32)

    @pl.when(i >= NIN + 2 * NC)
    def _():
        k = i - (NIN + 2 * NC)
        sl = pl.ds(k * RC, RC)
        blk = adj8v[sl, :].astype(BF16)
        acc4b = acc4t[:, sl].T
        t = jax.nn.relu(d2s[sl, :] * (acc4b + v3s[sl, :]) + bg1_ref[...])
        v4 = d2s[sl, :] * jnp.dot(t, wg2_ref[...], preferred_element_type=F32)
        v4s[sl, :] = v4

        @pl.when(k == 0)
        def _():
            acc5t[...] = jnp.zeros_like(acc5t)

        acc5t[...] += jax.lax.dot_general(v4.astype(BF16), blk, TDIMS,
                                          preferred_element_type=F32)

        @pl.when(k == NC - 1)
        def _():
            t2 = d2s[...] * (acc5t[...].T + v4s[...]) + bg2_ref[...]
            t3 = jnp.log(jnp.exp(t2) + 1.1)
            o = logits_s[...] * t3
            m = jnp.max(o, axis=1, keepdims=True)
            lse = m + jnp.log(jnp.sum(jnp.exp(o - m), axis=1, keepdims=True))
            out_ref[...] = o - lse


@jax.jit
def kernel(x, adj, W0, b0, W1, b1, Wg1, bg1, Wg2, bg2):
    D = x.shape[1]
    H = W0.shape[1]
    C = W1.shape[1]

    def strip(f):
        return pl.BlockSpec((R0, f),
                            lambda i: (jnp.minimum(i, NIN - 1), 0))

    def full(n, f):
        return pl.BlockSpec((n, f), lambda i: (0, 0))

    out = pl.pallas_call(
        _mega,
        grid=(NIN + 3 * NC,),
        in_specs=[strip(N), strip(D), full(D, H), full(1, H), full(H, C),
                  full(1, C), full(C, C), full(1, C), full(C, C),
                  full(1, C)],
        out_specs=full(N, C),
        out_shape=jax.ShapeDtypeStruct((N, C), F32),
        scratch_shapes=[
            pltpu.VMEM((N, N), jnp.int8),
            pltpu.VMEM((N, 1), F32),
            pltpu.VMEM((1, N), F32),
            pltpu.VMEM((N, 1), F32),
            pltpu.VMEM((N, H), BF16),
            pltpu.VMEM((N, C), F32),
            pltpu.VMEM((N, C), F32),
            pltpu.VMEM((N, C), F32),
            pltpu.VMEM((C, N), F32),
            pltpu.VMEM((N, C), F32),
            pltpu.VMEM((C, N), F32),
        ],
        compiler_params=pltpu.CompilerParams(
            vmem_limit_bytes=100 * 1024 * 1024),
    )(adj, x, W0, b0[None, :], W1, b1[None, :], Wg1, bg1[None, :], Wg2,
      bg2[None, :])

    return out


# speedup vs baseline: 1.5551x; 1.0190x over previous
import jax
import jax.numpy as jnp
from jax.experimental import pallas as pl
from jax.experimental.pallas import tpu as pltpu

N = 4096
R0 = 256
NIN = N // R0
RC = 2048
NC = N // RC
F32 = jnp.float32
BF16 = jnp.bfloat16
TDIMS = (((0,), (0,)), ((), ()))


def _mega(adj_ref, x_ref, w0_ref, b0_ref, w1_ref, b1_ref, wg1_ref, bg1_ref,
          wg2_ref, bg2_ref, out_ref,
          adj8v, d1s, css, d2s, v1s, v2s, logits_s, v3s, acc4t, v4s, acc5t):
    i = pl.program_id(0)

    @pl.when(i < NIN)
    def _():
        sl = pl.ds(i * R0, R0)
        blk = adj_ref[...]
        adj8v[sl, :] = blk.astype(jnp.int8)
        rs = jnp.sum(blk, axis=1, keepdims=True)
        d1 = (rs + 1.0) ** -0.5
        d1s[sl, :] = d1

        @pl.when(i == 0)
        def _():
            css[...] = jnp.zeros_like(css)

        css[...] += jnp.sum(blk, axis=0, keepdims=True)
        xw = jnp.dot(x_ref[...], w0_ref[...], preferred_element_type=F32)
        v1s[sl, :] = (d1 * xw).astype(BF16)

    @pl.when(jnp.logical_and(i >= NIN, i < NIN + NC))
    def _():
        k = i - NIN
        sl = pl.ds(k * RC, RC)

        @pl.when(k == 0)
        def _():
            d2s[...] = (css[...].T + 1.0) ** -0.5

        blk = adj8v[sl, :].astype(BF16)
        acc = jnp.dot(blk, v1s[...], preferred_element_type=F32)
        pre = acc + v1s[sl, :].astype(F32)
        d1 = d1s[sl, :]
        h1 = jax.nn.relu(d1 * pre + b0_ref[...])
        v2s[sl, :] = d1 * jnp.dot(h1, w1_ref[...], preferred_element_type=F32)

    @pl.when(jnp.logical_and(i >= NIN + NC, i < NIN + 2 * NC))
    def _():
        k = i - (NIN + NC)
        sl = pl.ds(k * RC, RC)
        blk = adj8v[sl, :].astype(BF16)
        acc = jnp.dot(blk, v2s[...].astype(BF16), preferred_element_type=F32)
        logits = d1s[sl, :] * (acc + v2s[sl, :]) + b1_ref[...]
        logits_s[sl, :] = logits
        v3 = d2s[sl, :] * jnp.dot(logits, wg1_ref[...],
                                  preferred_element_type=F32)
        v3s[sl, :] = v3

        @pl.when(k == 0)
        def _():
            acc4t[...] = jnp.zeros_like(acc4t)

        acc4t[...] += jax.lax.dot_general(v3.astype(BF16), blk, TDIMS,
                                          preferred_element_type=F32)

    @pl.when(i >= NIN + 2 * NC)
    def _():
        k = i - (NIN + 2 * NC)
        sl = pl.ds(k * RC, RC)
        blk = adj8v[sl, :].astype(BF16)
        acc4b = acc4t[:, sl].T
        t = jax.nn.relu(d2s[sl, :] * (acc4b + v3s[sl, :]) + bg1_ref[...])
        v4 = d2s[sl, :] * jnp.dot(t, wg2_ref[...], preferred_element_type=F32)
        v4s[sl, :] = v4

        @pl.when(k == 0)
        def _():
            acc5t[...] = jnp.zeros_like(acc5t)

        acc5t[...] += jax.lax.dot_general(v4.astype(BF16), blk, TDIMS,
                                          preferred_element_type=F32)

        @pl.when(k == NC - 1)
        def _():
            t2 = d2s[...] * (acc5t[...].T + v4s[...]) + bg2_ref[...]
            t3 = jnp.log(jnp.exp(t2) + 1.1)
            o = logits_s[...] * t3
            m = jnp.max(o, axis=1, keepdims=True)
            lse = m + jnp.log(jnp.sum(jnp.exp(o - m), axis=1, keepdims=True))
            out_ref[...] = o - lse


@jax.jit
def kernel(x, adj, W0, b0, W1, b1, Wg1, bg1, Wg2, bg2):
    D = x.shape[1]
    H = W0.shape[1]
    C = W1.shape[1]

    def strip(f):
        return pl.BlockSpec((R0, f),
                            lambda i: (jnp.minimum(i, NIN - 1), 0))

    def full(n, f):
        return pl.BlockSpec((n, f), lambda i: (0, 0))

    out = pl.pallas_call(
        _mega,
        grid=(NIN + 3 * NC,),
        in_specs=[strip(N), strip(D), full(D, H), full(1, H), full(H, C),
                  full(1, C), full(C, C), full(1, C), full(C, C),
                  full(1, C)],
        out_specs=full(N, C),
        out_shape=jax.ShapeDtypeStruct((N, C), F32),
        scratch_shapes=[
            pltpu.VMEM((N, N), jnp.int8),
            pltpu.VMEM((N, 1), F32),
            pltpu.VMEM((1, N), F32),
            pltpu.VMEM((N, 1), F32),
            pltpu.VMEM((N, H), BF16),
            pltpu.VMEM((N, C), F32),
            pltpu.VMEM((N, C), F32),
            pltpu.VMEM((N, C), F32),
            pltpu.VMEM((C, N), F32),
            pltpu.VMEM((N, C), F32),
            pltpu.VMEM((C, N), F32),
        ],
        compiler_params=pltpu.CompilerParams(
            vmem_limit_bytes=100 * 1024 * 1024),
    )(adj, x, W0, b0[None, :], W1, b1[None, :], Wg1, bg1[None, :], Wg2,
      bg2[None, :])

    return out


# speedup vs baseline: 1.6319x; 1.0494x over previous
import jax
import jax.numpy as jnp
from jax.experimental import pallas as pl
from jax.experimental.pallas import tpu as pltpu

N = 4096
R0 = 512
NIN = N // R0
RC = 2048
NC = N // RC
F32 = jnp.float32
BF16 = jnp.bfloat16
TDIMS = (((0,), (0,)), ((), ()))


def _mega(adj_ref, x_ref, w0_ref, b0_ref, w1_ref, b1_ref, wg1_ref, bg1_ref,
          wg2_ref, bg2_ref, out_ref,
          adj8v, d1s, css, d2s, v1s, v2s, logits_s, v3s, acc4t, v4s, acc5t):
    i = pl.program_id(0)

    @pl.when(i < NIN)
    def _():
        sl = pl.ds(i * R0, R0)
        blk = adj_ref[...]
        adj8v[sl, :] = blk.astype(jnp.int8)
        rs = jnp.sum(blk, axis=1, keepdims=True)
        d1 = (rs + 1.0) ** -0.5
        d1s[sl, :] = d1

        @pl.when(i == 0)
        def _():
            css[...] = jnp.zeros_like(css)

        css[...] += jnp.sum(blk, axis=0, keepdims=True)
        xw = jnp.dot(x_ref[...], w0_ref[...], preferred_element_type=F32)
        v1s[sl, :] = (d1 * xw).astype(BF16)

    @pl.when(jnp.logical_and(i >= NIN, i < NIN + NC))
    def _():
        k = i - NIN
        sl = pl.ds(k * RC, RC)

        @pl.when(k == 0)
        def _():
            d2s[...] = (css[...].T + 1.0) ** -0.5

        blk = adj8v[sl, :].astype(BF16)
        acc = jnp.dot(blk, v1s[...], preferred_element_type=F32)
        pre = acc + v1s[sl, :].astype(F32)
        d1 = d1s[sl, :]
        h1 = jax.nn.relu(d1 * pre + b0_ref[...])
        v2s[sl, :] = d1 * jnp.dot(h1, w1_ref[...], preferred_element_type=F32)

    @pl.when(jnp.logical_and(i >= NIN + NC, i < NIN + 2 * NC))
    def _():
        k = i - (NIN + NC)
        sl = pl.ds(k * RC, RC)
        blk = adj8v[sl, :].astype(BF16)
        acc = jnp.dot(blk, v2s[...].astype(BF16), preferred_element_type=F32)
        logits = d1s[sl, :] * (acc + v2s[sl, :]) + b1_ref[...]
        logits_s[sl, :] = logits
        v3 = d2s[sl, :] * jnp.dot(logits, wg1_ref[...],
                                  preferred_element_type=F32)
        v3s[sl, :] = v3

        @pl.when(k == 0)
        def _():
            acc4t[...] = jnp.zeros_like(acc4t)

        acc4t[...] += jax.lax.dot_general(v3.astype(BF16), blk, TDIMS,
                                          preferred_element_type=F32)

    @pl.when(i >= NIN + 2 * NC)
    def _():
        k = i - (NIN + 2 * NC)
        sl = pl.ds(k * RC, RC)
        blk = adj8v[sl, :].astype(BF16)
        acc4b = acc4t[:, sl].T
        t = jax.nn.relu(d2s[sl, :] * (acc4b + v3s[sl, :]) + bg1_ref[...])
        v4 = d2s[sl, :] * jnp.dot(t, wg2_ref[...], preferred_element_type=F32)
        v4s[sl, :] = v4

        @pl.when(k == 0)
        def _():
            acc5t[...] = jnp.zeros_like(acc5t)

        acc5t[...] += jax.lax.dot_general(v4.astype(BF16), blk, TDIMS,
                                          preferred_element_type=F32)

        @pl.when(k == NC - 1)
        def _():
            t2 = d2s[...] * (acc5t[...].T + v4s[...]) + bg2_ref[...]
            t3 = jnp.log(jnp.exp(t2) + 1.1)
            o = logits_s[...] * t3
            m = jnp.max(o, axis=1, keepdims=True)
            lse = m + jnp.log(jnp.sum(jnp.exp(o - m), axis=1, keepdims=True))
            out_ref[...] = o - lse


@jax.jit
def kernel(x, adj, W0, b0, W1, b1, Wg1, bg1, Wg2, bg2):
    D = x.shape[1]
    H = W0.shape[1]
    C = W1.shape[1]

    def strip(f):
        return pl.BlockSpec((R0, f),
                            lambda i: (jnp.minimum(i, NIN - 1), 0))

    def full(n, f):
        return pl.BlockSpec((n, f), lambda i: (0, 0))

    out = pl.pallas_call(
        _mega,
        grid=(NIN + 3 * NC,),
        in_specs=[strip(N), strip(D), full(D, H), full(1, H), full(H, C),
                  full(1, C), full(C, C), full(1, C), full(C, C),
                  full(1, C)],
        out_specs=full(N, C),
        out_shape=jax.ShapeDtypeStruct((N, C), F32),
        scratch_shapes=[
            pltpu.VMEM((N, N), jnp.int8),
            pltpu.VMEM((N, 1), F32),
            pltpu.VMEM((1, N), F32),
            pltpu.VMEM((N, 1), F32),
            pltpu.VMEM((N, H), BF16),
            pltpu.VMEM((N, C), F32),
            pltpu.VMEM((N, C), F32),
            pltpu.VMEM((N, C), F32),
            pltpu.VMEM((C, N), F32),
            pltpu.VMEM((N, C), F32),
            pltpu.VMEM((C, N), F32),
        ],
        compiler_params=pltpu.CompilerParams(
            vmem_limit_bytes=100 * 1024 * 1024),
    )(adj, x, W0, b0[None, :], W1, b1[None, :], Wg1, bg1[None, :], Wg2,
      bg2[None, :])

    return out
